# Initial kernel scaffold; baseline (speedup 1.0000x reference)
import jax
import jax.numpy as jnp
from jax.experimental import pallas as pl


def kernel(complex_x, complex_pos, complex_edge_index, complex_edge_attr, emb_in_w, emb_in_b, emb_out_w, emb_out_b, edge_w1, edge_b1, edge_w2, edge_b2, node_w1, node_b1, node_w2, node_b2, coord_w1, coord_b1, coord_w2):
    raise NotImplementedError("write your pallas kernel here")


# speedup vs baseline: 2.5954x; 2.5954x over previous
import functools

import jax
import jax.numpy as jnp
from jax import lax
from jax.experimental import pallas as pl
from jax.experimental.pallas import tpu as pltpu
from jax.experimental.pallas import tpu_sc as plsc

N = 10000
NPAD = 10240
E = 320000
EPAD = 327680
ROWS = EPAD // 128
HID = 128
EDIM = 16
NC = 2
NS = 16
NW = NC * NS
RPW = ROWS // NW
K = 2
NCHUNK = RPW // K
BE = 1024
BN = 1024
F32 = jnp.float32


def _silu(x):
    return x * jax.nn.sigmoid(x)


def _k0_body(x_ref, wi_ref, bi_ref, wa_ref, wb_ref, h_ref, ha_ref, hb_ref):
    h = x_ref[...] @ wi_ref[...] + bi_ref[...]
    h_ref[...] = h
    ha_ref[...] = h @ wa_ref[...]
    hb_ref[...] = h @ wb_ref[...]


def _edge_body(ha_ref, hb_ref, d0_ref, d1_ref, d2_ref, rad_ref, ea_ref,
               wr_ref, we_ref, b1_ref, w2_ref, b2_ref,
               c1_ref, cb1_ref, c2r_ref, m_ref, t0_ref, t1o_ref, t2_ref):
    radial = rad_ref[...]
    t1 = (ha_ref[...] + hb_ref[...] + radial * wr_ref[...]
          + ea_ref[...] @ we_ref[...] + b1_ref[...])
    m1 = _silu(t1)
    m2 = _silu(m1 @ w2_ref[...] + b2_ref[...])
    cm = _silu(m2 @ c1_ref[...] + cb1_ref[...])
    m_ref[...] = m2
    sca = jnp.sum(cm * c2r_ref[...], axis=1, keepdims=True)
    t0_ref[...] = d0_ref[...] * sca
    t1o_ref[...] = d1_ref[...] * sca
    t2_ref[...] = d2_ref[...] * sca


def _coord_update(xt, a4):
    ct = jnp.sum(a4, axis=0)
    cnt = jnp.clip(ct[3:4, :], 1.0, None)
    comp = lax.broadcasted_iota(jnp.int32, ct.shape, 0)
    return xt + jnp.where(comp < 3, ct / cnt, 0.0)


def _node_mid_body(h_ref, m0_ref, m1_ref, a4_ref, xt_ref,
                   w1h_ref, w1m_ref, nb1_ref, w2_ref, nb2_ref,
                   wa_ref, wb_ref, hn_ref, xn_ref, ha_ref, hb_ref):
    magg = m0_ref[...] + m1_ref[...]
    o = _silu(h_ref[...] @ w1h_ref[...] + magg @ w1m_ref[...] + nb1_ref[...])
    o = o @ w2_ref[...] + nb2_ref[...]
    hn = h_ref[...] + o
    hn_ref[...] = hn
    xn_ref[...] = _coord_update(xt_ref[...], a4_ref[...])
    ha_ref[...] = hn @ wa_ref[...]
    hb_ref[...] = hn @ wb_ref[...]


def _node_fin_body(h_ref, m0_ref, m1_ref, a4_ref, xt_ref,
                   w1h_ref, w1m_ref, nb1_ref, w2_ref, nb2_ref,
                   wo_ref, bo_ref, hout_ref, xn_ref):
    magg = m0_ref[...] + m1_ref[...]
    o = _silu(h_ref[...] @ w1h_ref[...] + magg @ w1m_ref[...] + nb1_ref[...])
    o = o @ w2_ref[...] + nb2_ref[...]
    hn = h_ref[...] + o
    hout_ref[...] = hn @ wo_ref[...] + bo_ref[...]
    xn_ref[...] = _coord_update(xt_ref[...], a4_ref[...])


def _gather_body(tha, thb, xf, rown, coln, oha, ohb, od0, od1, od2, orad,
                 ridx, cidx, cix, gx, bha, bhb, bdd, brad, sem):
    c = lax.axis_index("c")
    s = lax.axis_index("s")
    wid = s * NC + c
    base = wid * RPW

    def chunk(i, carry):
        r0 = base + i * K
        pltpu.sync_copy(rown.at[pl.ds(r0, K)], ridx)
        pltpu.sync_copy(coln.at[pl.ds(r0, K)], cidx)
        descs = []
        for j in range(K):
            descs.append(pltpu.async_copy(tha.at[ridx.at[j]], bha.at[j], sem))
            descs.append(pltpu.async_copy(thb.at[cidx.at[j]], bhb.at[j], sem))
        for j in range(K):
            for t in range(8):
                sl = pl.ds(t * 16, 16)
                ir = ridx[j, sl]
                ic = cidx[j, sl]
                for comp in range(3):
                    off = jnp.int32(comp * NPAD)
                    cix[comp, j, sl] = ir + off
                    cix[3 + comp, j, sl] = ic + off
        for q in range(6):
            for j in range(K):
                descs.append(
                    pltpu.async_copy(xf.at[cix.at[q, j]], gx.at[q, j], sem))
        for d in descs:
            d.wait()
        for j in range(K):
            for t in range(8):
                sl = pl.ds(t * 16, 16)
                rad = jnp.zeros((16,), F32)
                for comp in range(3):
                    dv = gx[comp, j, sl] - gx[3 + comp, j, sl]
                    bdd[comp, j, sl] = dv
                    rad = rad + dv * dv
                brad[j, sl] = rad
        pltpu.sync_copy(bha, oha.at[pl.ds(r0, K)])
        pltpu.sync_copy(bhb, ohb.at[pl.ds(r0, K)])
        pltpu.sync_copy(bdd.at[0], od0.at[pl.ds(r0, K)])
        pltpu.sync_copy(bdd.at[1], od1.at[pl.ds(r0, K)])
        pltpu.sync_copy(bdd.at[2], od2.at[pl.ds(r0, K)])
        pltpu.sync_copy(brad, orad.at[pl.ds(r0, K)])
        return carry

    lax.fori_loop(0, NCHUNK, chunk, 0)


def _scatter_body(zm, z4, rown, m3, t0, t1, t2, outm, outt,
                  ridx, mbuf, tbuf, tix, ones, acc4, accm):
    c = lax.axis_index("c")
    s = lax.axis_index("s")
    base = (s * NC + c) * RPW
    stripe = NPAD // NS
    stripe4 = 4 * NPAD // NS
    pltpu.sync_copy(zm, accm.at[pl.ds(s * stripe, stripe)])
    pltpu.sync_copy(z4, acc4.at[pl.ds(s * stripe4, stripe4)])
    for t in range(8):
        ones[pl.ds(t * 16, 16)] = jnp.ones((16,), F32)
    plsc.subcore_barrier()

    def chunk(i, carry):
        r0 = base + i * K
        pltpu.sync_copy(rown.at[pl.ds(r0, K)], ridx)
        pltpu.sync_copy(m3.at[pl.ds(r0, K)], mbuf)
        pltpu.sync_copy(t0.at[pl.ds(r0, K)], tbuf.at[0])
        pltpu.sync_copy(t1.at[pl.ds(r0, K)], tbuf.at[1])
        pltpu.sync_copy(t2.at[pl.ds(r0, K)], tbuf.at[2])
        for j in range(K):
            for t in range(8):
                sl = pl.ds(t * 16, 16)
                ir = ridx[j, sl]
                for q in range(4):
                    tix[q, j, sl] = ir + jnp.int32(q * NPAD)
        for j in range(K):
            pltpu.sync_copy(mbuf.at[j], accm.at[ridx.at[j]], add=True)
            for q in range(3):
                pltpu.sync_copy(tbuf.at[q, j], acc4.at[tix.at[q, j]],
                                add=True)
            pltpu.sync_copy(ones, acc4.at[tix.at[3, j]], add=True)
        return carry

    lax.fori_loop(0, NCHUNK, chunk, 0)
    plsc.subcore_barrier()

    @pl.when(s == 0)
    def _():
        pltpu.sync_copy(accm, outm.at[c])
        pltpu.sync_copy(acc4, outt.at[c])


def _full(shape):
    return pl.BlockSpec(shape, lambda i: (0,) * len(shape))


def _k0(x, wi, bi, wa, wb):
    grid = (NPAD // BN,)
    blk = pl.BlockSpec((BN, HID), lambda i: (i, 0))
    return pl.pallas_call(
        _k0_body,
        grid=grid,
        in_specs=[blk, _full((HID, HID)), _full((1, HID)),
                  _full((HID, HID)), _full((HID, HID))],
        out_specs=[blk, blk, blk],
        out_shape=[jax.ShapeDtypeStruct((NPAD, HID), F32)] * 3,
    )(x, wi, bi, wa, wb)


def _edge(ha, hb, d0, d1, d2, rad, ea, wr, we, b1, w2, b2, c1, cb1, c2r):
    grid = (EPAD // BE,)
    blk = pl.BlockSpec((BE, HID), lambda i: (i, 0))
    blk1 = pl.BlockSpec((BE, 1), lambda i: (i, 0))
    blke = pl.BlockSpec((BE, EDIM), lambda i: (i, 0))
    return pl.pallas_call(
        _edge_body,
        grid=grid,
        in_specs=[blk, blk, blk1, blk1, blk1, blk1, blke,
                  _full((1, HID)), _full((EDIM, HID)), _full((1, HID)),
                  _full((HID, HID)), _full((1, HID)),
                  _full((HID, HID)), _full((1, HID)), _full((1, HID))],
        out_specs=[blk, blk1, blk1, blk1],
        out_shape=[jax.ShapeDtypeStruct((EPAD, HID), F32),
                   jax.ShapeDtypeStruct((EPAD, 1), F32),
                   jax.ShapeDtypeStruct((EPAD, 1), F32),
                   jax.ShapeDtypeStruct((EPAD, 1), F32)],
    )(ha, hb, d0, d1, d2, rad, ea, wr, we, b1, w2, b2, c1, cb1, c2r)


def _node_mid(h, m0, m1, a4, xt, w1h, w1m, nb1, w2, nb2, wa, wb):
    grid = (NPAD // BN,)
    blk = pl.BlockSpec((BN, HID), lambda i: (i, 0))
    blkx = pl.BlockSpec((4, BN), lambda i: (0, i))
    blka = pl.BlockSpec((NC, 4, BN), lambda i: (0, 0, i))
    return pl.pallas_call(
        _node_mid_body,
        grid=grid,
        in_specs=[blk, blk, blk, blka, blkx,
                  _full((HID, HID)), _full((HID, HID)), _full((1, HID)),
                  _full((HID, HID)), _full((1, HID)),
                  _full((HID, HID)), _full((HID, HID))],
        out_specs=[blk, blkx, blk, blk],
        out_shape=[jax.ShapeDtypeStruct((NPAD, HID), F32),
                   jax.ShapeDtypeStruct((4, NPAD), F32),
                   jax.ShapeDtypeStruct((NPAD, HID), F32),
                   jax.ShapeDtypeStruct((NPAD, HID), F32)],
    )(h, m0, m1, a4, xt, w1h, w1m, nb1, w2, nb2, wa, wb)


def _node_fin(h, m0, m1, a4, xt, w1h, w1m, nb1, w2, nb2, wo, bo):
    grid = (NPAD // BN,)
    blk = pl.BlockSpec((BN, HID), lambda i: (i, 0))
    blkx = pl.BlockSpec((4, BN), lambda i: (0, i))
    blka = pl.BlockSpec((NC, 4, BN), lambda i: (0, 0, i))
    return pl.pallas_call(
        _node_fin_body,
        grid=grid,
        in_specs=[blk, blk, blk, blka, blkx,
                  _full((HID, HID)), _full((HID, HID)), _full((1, HID)),
                  _full((HID, HID)), _full((1, HID)),
                  _full((HID, HID)), _full((1, HID))],
        out_specs=[blk, blkx],
        out_shape=[jax.ShapeDtypeStruct((NPAD, HID), F32),
                   jax.ShapeDtypeStruct((4, NPAD), F32)],
    )(h, m0, m1, a4, xt, w1h, w1m, nb1, w2, nb2, wo, bo)


def _sc_gather(tha, thb, xf, rown, coln):
    mesh = plsc.VectorSubcoreMesh(core_axis_name="c", subcore_axis_name="s")
    fn = pl.kernel(
        _gather_body,
        out_type=[jax.ShapeDtypeStruct((ROWS, 128, HID), F32),
                  jax.ShapeDtypeStruct((ROWS, 128, HID), F32),
                  jax.ShapeDtypeStruct((ROWS, 128), F32),
                  jax.ShapeDtypeStruct((ROWS, 128), F32),
                  jax.ShapeDtypeStruct((ROWS, 128), F32),
                  jax.ShapeDtypeStruct((ROWS, 128), F32)],
        mesh=mesh,
        scratch_types=[pltpu.VMEM((K, 128), jnp.int32),
                       pltpu.VMEM((K, 128), jnp.int32),
                       pltpu.VMEM((6, K, 128), jnp.int32),
                       pltpu.VMEM((6, K, 128), F32),
                       pltpu.VMEM((K, 128, HID), F32),
                       pltpu.VMEM((K, 128, HID), F32),
                       pltpu.VMEM((3, K, 128), F32),
                       pltpu.VMEM((K, 128), F32),
                       pltpu.SemaphoreType.DMA],
    )
    return fn(tha, thb, xf, rown, coln)


def _sc_scatter(zm, z4, rown, m3, t0, t1, t2):
    mesh = plsc.VectorSubcoreMesh(core_axis_name="c", subcore_axis_name="s")
    fn = pl.kernel(
        _scatter_body,
        out_type=[jax.ShapeDtypeStruct((NC, NPAD, HID), F32),
                  jax.ShapeDtypeStruct((NC, 4 * NPAD), F32)],
        mesh=mesh,
        scratch_types=[pltpu.VMEM((K, 128), jnp.int32),
                       pltpu.VMEM((K, 128, HID), F32),
                       pltpu.VMEM((3, K, 128), F32),
                       pltpu.VMEM((4, K, 128), jnp.int32),
                       pltpu.VMEM((128,), F32),
                       pltpu.VMEM_SHARED((4 * NPAD,), F32),
                       pltpu.VMEM_SHARED((NPAD, HID), F32)],
    )
    return fn(zm, z4, rown, m3, t0, t1, t2)


def kernel(complex_x, complex_pos, complex_edge_index, complex_edge_attr,
           emb_in_w, emb_in_b, emb_out_w, emb_out_b,
           edge_w1, edge_b1, edge_w2, edge_b2,
           node_w1, node_b1, node_w2, node_b2,
           coord_w1, coord_b1, coord_w2):
    row = complex_edge_index[0]
    col = complex_edge_index[1]
    pad_ids = (N + (jnp.arange(EPAD - E, dtype=jnp.int32) % (NPAD - N))
               ).astype(jnp.int32)
    rown = jnp.concatenate([row, pad_ids]).reshape(ROWS, 128)
    coln = jnp.concatenate([col, pad_ids]).reshape(ROWS, 128)
    x0 = jnp.pad(complex_pos.T, ((0, 1), (0, NPAD - N)))
    xin = jnp.pad(complex_x, ((0, NPAD - N), (0, 0)))
    eap = jnp.pad(complex_edge_attr, ((0, EPAD - E), (0, 0)))
    zm = jnp.zeros((NPAD // NS, HID), F32)
    z4 = jnp.zeros((4 * NPAD // NS,), F32)

    def wsplit(l):
        w1 = edge_w1[l]
        return (w1[:HID], w1[HID:2 * HID], w1[2 * HID:2 * HID + 1],
                w1[2 * HID + 1:])

    wa0, wb0, _, _ = wsplit(0)
    h, ha, hb = _k0(xin, emb_in_w, emb_in_b[None], wa0, wb0)
    x = x0
    for l in range(2):
        _, _, wr, we = wsplit(l)
        hag, hbg, gd0, gd1, gd2, radg = _sc_gather(
            ha, hb, x.reshape(4 * NPAD), rown, coln)
        m2, tr0, tr1, tr2 = _edge(
            hag.reshape(EPAD, HID), hbg.reshape(EPAD, HID),
            gd0.reshape(EPAD, 1), gd1.reshape(EPAD, 1), gd2.reshape(EPAD, 1),
            radg.reshape(EPAD, 1), eap,
            wr, we, edge_b1[l][None], edge_w2[l], edge_b2[l][None],
            coord_w1[l], coord_b1[l][None], coord_w2[l].reshape(1, HID))
        accm, acct = _sc_scatter(zm, z4, rown,
                                 m2.reshape(ROWS, 128, HID),
                                 tr0.reshape(ROWS, 128),
                                 tr1.reshape(ROWS, 128),
                                 tr2.reshape(ROWS, 128))
        acct = acct.reshape(NC, 4, NPAD)
        w1h = node_w1[l][:HID]
        w1m = node_w1[l][HID:]
        if l == 0:
            wa1, wb1, _, _ = wsplit(1)
            h, x, ha, hb = _node_mid(
                h, accm[0], accm[1], acct, x,
                w1h, w1m, node_b1[l][None], node_w2[l], node_b2[l][None],
                wa1, wb1)
        else:
            hout, x = _node_fin(
                h, accm[0], accm[1], acct, x,
                w1h, w1m, node_b1[l][None], node_w2[l], node_b2[l][None],
                emb_out_w, emb_out_b[None])
    return hout[:N], x[:3, :N].T


# speedup vs baseline: 2.8764x; 1.1083x over previous
import functools

import jax
import jax.numpy as jnp
from jax import lax
from jax.experimental import pallas as pl
from jax.experimental.pallas import tpu as pltpu
from jax.experimental.pallas import tpu_sc as plsc

N = 10000
NPAD = 10240
E = 320000
EPAD = 327680
ROWS = EPAD // 128
HID = 128
EDIM = 16
NC = 2
NS = 16
NW = NC * NS
RPW = ROWS // NW
K = 2
NCHUNK = RPW // K
BE = 1024
BN = 1024
F32 = jnp.float32


def _silu(x):
    return x * jax.nn.sigmoid(x)


def _k0_body(x_ref, wi_ref, bi_ref, wa_ref, wb_ref, h_ref, ha_ref, hb_ref):
    h = x_ref[...] @ wi_ref[...] + bi_ref[...]
    h_ref[...] = h
    ha_ref[...] = h @ wa_ref[...]
    hb_ref[...] = h @ wb_ref[...]


def _edge_body(ha_ref, hb_ref, d0_ref, d1_ref, d2_ref, rad_ref, ea_ref,
               wr_ref, we_ref, b1_ref, w2_ref, b2_ref,
               c1_ref, cb1_ref, c2r_ref, m_ref, t0_ref, t1o_ref, t2_ref):
    radial = rad_ref[...]
    t1 = (ha_ref[...] + hb_ref[...] + radial * wr_ref[...]
          + ea_ref[...] @ we_ref[...] + b1_ref[...])
    m1 = _silu(t1)
    m2 = _silu(m1 @ w2_ref[...] + b2_ref[...])
    cm = _silu(m2 @ c1_ref[...] + cb1_ref[...])
    m_ref[...] = m2
    sca = jnp.sum(cm * c2r_ref[...], axis=1, keepdims=True)
    t0_ref[...] = d0_ref[...] * sca
    t1o_ref[...] = d1_ref[...] * sca
    t2_ref[...] = d2_ref[...] * sca


def _coord_update(xt, a4):
    ct = jnp.sum(a4, axis=0)
    cnt = jnp.clip(ct[3:4, :], 1.0, None)
    comp = lax.broadcasted_iota(jnp.int32, ct.shape, 0)
    return xt + jnp.where(comp < 3, ct / cnt, 0.0)


def _node_mid_body(h_ref, m0_ref, m1_ref, a4_ref, xt_ref,
                   w1h_ref, w1m_ref, nb1_ref, w2_ref, nb2_ref,
                   wa_ref, wb_ref, hn_ref, xn_ref, ha_ref, hb_ref):
    magg = m0_ref[...] + m1_ref[...]
    o = _silu(h_ref[...] @ w1h_ref[...] + magg @ w1m_ref[...] + nb1_ref[...])
    o = o @ w2_ref[...] + nb2_ref[...]
    hn = h_ref[...] + o
    hn_ref[...] = hn
    xn_ref[...] = _coord_update(xt_ref[...], a4_ref[...])
    ha_ref[...] = hn @ wa_ref[...]
    hb_ref[...] = hn @ wb_ref[...]


def _node_fin_body(h_ref, m0_ref, m1_ref, a4_ref, xt_ref,
                   w1h_ref, w1m_ref, nb1_ref, w2_ref, nb2_ref,
                   wo_ref, bo_ref, hout_ref, xn_ref):
    magg = m0_ref[...] + m1_ref[...]
    o = _silu(h_ref[...] @ w1h_ref[...] + magg @ w1m_ref[...] + nb1_ref[...])
    o = o @ w2_ref[...] + nb2_ref[...]
    hn = h_ref[...] + o
    hout_ref[...] = hn @ wo_ref[...] + bo_ref[...]
    xn_ref[...] = _coord_update(xt_ref[...], a4_ref[...])


def _gather_body(tha, thb, xf, rown, coln, oha, ohb, od0, od1, od2, orad,
                 ridx, cidx, cix, gx, bha, bhb, bdd, brad,
                 semI, semG, semW):
    c = lax.axis_index("c")
    s = lax.axis_index("s")
    base = (s * NC + c) * RPW

    def fire_idx(slot, r):
        pltpu.async_copy(rown.at[pl.ds(r, 1)], ridx.at[slot], semI)
        pltpu.async_copy(coln.at[pl.ds(r, 1)], cidx.at[slot], semI)

    def drain_idx(slot, r):
        pltpu.make_async_copy(rown.at[pl.ds(r, 1)], ridx.at[slot],
                              semI).wait()
        pltpu.make_async_copy(coln.at[pl.ds(r, 1)], cidx.at[slot],
                              semI).wait()

    def compute_cix(slot):
        for t in range(8):
            sl = pl.ds(t * 16, 16)
            ir = ridx[slot, 0, sl]
            ic = cidx[slot, 0, sl]
            for comp in range(3):
                off = jnp.int32(comp * NPAD)
                cix[slot, comp, sl] = ir + off
                cix[slot, 3 + comp, sl] = ic + off

    def fire_gathers(slot):
        pltpu.async_copy(tha.at[ridx.at[slot, 0]], bha.at[slot], semG)
        pltpu.async_copy(thb.at[cidx.at[slot, 0]], bhb.at[slot], semG)
        for q in range(6):
            pltpu.async_copy(xf.at[cix.at[slot, q]], gx.at[slot, q], semG)

    def drain_gathers(slot):
        pltpu.make_async_copy(tha.at[ridx.at[slot, 0]], bha.at[slot],
                              semG).wait()
        pltpu.make_async_copy(thb.at[cidx.at[slot, 0]], bhb.at[slot],
                              semG).wait()
        for q in range(6):
            pltpu.make_async_copy(xf.at[cix.at[slot, q]], gx.at[slot, q],
                                  semG).wait()

    def compute_diff(slot):
        for t in range(8):
            sl = pl.ds(t * 16, 16)
            rad = jnp.zeros((16,), F32)
            for comp in range(3):
                dv = gx[slot, comp, sl] - gx[slot, 3 + comp, sl]
                bdd[slot, comp, sl] = dv
                rad = rad + dv * dv
            brad[slot, sl] = rad

    def fire_writes(slot, r):
        pltpu.async_copy(bha.at[slot], oha.at[r], semW)
        pltpu.async_copy(bhb.at[slot], ohb.at[r], semW)
        pltpu.async_copy(bdd.at[slot, 0], od0.at[r], semW)
        pltpu.async_copy(bdd.at[slot, 1], od1.at[r], semW)
        pltpu.async_copy(bdd.at[slot, 2], od2.at[r], semW)
        pltpu.async_copy(brad.at[slot], orad.at[r], semW)

    def drain_writes(slot, r):
        pltpu.make_async_copy(bha.at[slot], oha.at[r], semW).wait()
        pltpu.make_async_copy(bhb.at[slot], ohb.at[r], semW).wait()
        pltpu.make_async_copy(bdd.at[slot, 0], od0.at[r], semW).wait()
        pltpu.make_async_copy(bdd.at[slot, 1], od1.at[r], semW).wait()
        pltpu.make_async_copy(bdd.at[slot, 2], od2.at[r], semW).wait()
        pltpu.make_async_copy(brad.at[slot], orad.at[r], semW).wait()

    pltpu.sync_copy(rown.at[pl.ds(base, 1)], ridx.at[0])
    pltpu.sync_copy(coln.at[pl.ds(base, 1)], cidx.at[0])
    compute_cix(0)
    fire_gathers(0)
    fire_idx(1, base + 1)

    def pair(i, carry):
        for b in range(2):
            ci = 2 * i + b
            r0 = base + ci
            nb = 1 - b

            @pl.when(ci >= 1)
            def _():
                drain_writes(nb, r0 - 1)
            drain_idx(nb, base + ((ci + 1) % RPW))
            compute_cix(nb)
            fire_gathers(nb)
            drain_gathers(b)
            fire_idx(b, base + ((ci + 2) % RPW))
            compute_diff(b)
            fire_writes(b, r0)
        return carry

    lax.fori_loop(0, RPW // 2, pair, 0)
    drain_gathers(0)
    drain_idx(1, base)
    drain_writes(1, base + RPW - 1)


def _scatter_body(zm, z4, rown, m3, t0, t1, t2, outm, outt,
                  ridx, mbuf, tbuf, tix, ones, acc4, accm, semL, semS):
    c = lax.axis_index("c")
    s = lax.axis_index("s")
    base = (s * NC + c) * RPW
    stripe = NPAD // NS
    stripe4 = 4 * NPAD // NS
    pltpu.sync_copy(zm, accm.at[pl.ds(s * stripe, stripe)])
    pltpu.sync_copy(z4, acc4.at[pl.ds(s * stripe4, stripe4)])
    for t in range(8):
        ones[pl.ds(t * 16, 16)] = jnp.ones((16,), F32)
    plsc.subcore_barrier()

    def fire_loads(slot, r):
        pltpu.async_copy(rown.at[pl.ds(r, 1)], ridx.at[slot], semL)
        pltpu.async_copy(m3.at[r], mbuf.at[slot], semL)
        pltpu.async_copy(t0.at[r], tbuf.at[slot, 0], semL)
        pltpu.async_copy(t1.at[r], tbuf.at[slot, 1], semL)
        pltpu.async_copy(t2.at[r], tbuf.at[slot, 2], semL)

    def drain_loads(slot, r):
        pltpu.make_async_copy(rown.at[pl.ds(r, 1)], ridx.at[slot],
                              semL).wait()
        pltpu.make_async_copy(m3.at[r], mbuf.at[slot], semL).wait()
        pltpu.make_async_copy(t0.at[r], tbuf.at[slot, 0], semL).wait()
        pltpu.make_async_copy(t1.at[r], tbuf.at[slot, 1], semL).wait()
        pltpu.make_async_copy(t2.at[r], tbuf.at[slot, 2], semL).wait()

    def fire_scatters(slot):
        pltpu.async_copy(mbuf.at[slot], accm.at[ridx.at[slot, 0]], semS,
                         add=True)
        for q in range(3):
            pltpu.async_copy(tbuf.at[slot, q], acc4.at[tix.at[slot, q]],
                             semS, add=True)
        pltpu.async_copy(ones, acc4.at[tix.at[slot, 3]], semS, add=True)

    def drain_scatters(slot):
        pltpu.make_async_copy(mbuf.at[slot], accm.at[ridx.at[slot, 0]],
                              semS).wait()
        for q in range(3):
            pltpu.make_async_copy(tbuf.at[slot, q],
                                  acc4.at[tix.at[slot, q]], semS).wait()
        pltpu.make_async_copy(ones, acc4.at[tix.at[slot, 3]], semS).wait()

    fire_loads(0, base)

    def pair(i, carry):
        for b in range(2):
            ci = 2 * i + b
            nb = 1 - b

            @pl.when(ci >= 1)
            def _():
                drain_scatters(nb)
            fire_loads(nb, base + ((ci + 1) % RPW))
            drain_loads(b, base + ci)
            for t in range(8):
                sl = pl.ds(t * 16, 16)
                ir = ridx[b, 0, sl]
                for q in range(4):
                    tix[b, q, sl] = ir + jnp.int32(q * NPAD)
            fire_scatters(b)
        return carry

    lax.fori_loop(0, RPW // 2, pair, 0)
    drain_scatters(1)
    drain_loads(0, base)
    plsc.subcore_barrier()

    @pl.when(s == 0)
    def _():
        pltpu.sync_copy(accm, outm.at[c])
        pltpu.sync_copy(acc4, outt.at[c])


def _full(shape):
    return pl.BlockSpec(shape, lambda i: (0,) * len(shape))


def _k0(x, wi, bi, wa, wb):
    grid = (NPAD // BN,)
    blk = pl.BlockSpec((BN, HID), lambda i: (i, 0))
    return pl.pallas_call(
        _k0_body,
        grid=grid,
        in_specs=[blk, _full((HID, HID)), _full((1, HID)),
                  _full((HID, HID)), _full((HID, HID))],
        out_specs=[blk, blk, blk],
        out_shape=[jax.ShapeDtypeStruct((NPAD, HID), F32)] * 3,
    )(x, wi, bi, wa, wb)


def _edge(ha, hb, d0, d1, d2, rad, ea, wr, we, b1, w2, b2, c1, cb1, c2r):
    grid = (EPAD // BE,)
    blk = pl.BlockSpec((BE, HID), lambda i: (i, 0))
    blk1 = pl.BlockSpec((BE, 1), lambda i: (i, 0))
    blke = pl.BlockSpec((BE, EDIM), lambda i: (i, 0))
    return pl.pallas_call(
        _edge_body,
        grid=grid,
        in_specs=[blk, blk, blk1, blk1, blk1, blk1, blke,
                  _full((1, HID)), _full((EDIM, HID)), _full((1, HID)),
                  _full((HID, HID)), _full((1, HID)),
                  _full((HID, HID)), _full((1, HID)), _full((1, HID))],
        out_specs=[blk, blk1, blk1, blk1],
        out_shape=[jax.ShapeDtypeStruct((EPAD, HID), F32),
                   jax.ShapeDtypeStruct((EPAD, 1), F32),
                   jax.ShapeDtypeStruct((EPAD, 1), F32),
                   jax.ShapeDtypeStruct((EPAD, 1), F32)],
    )(ha, hb, d0, d1, d2, rad, ea, wr, we, b1, w2, b2, c1, cb1, c2r)


def _node_mid(h, m0, m1, a4, xt, w1h, w1m, nb1, w2, nb2, wa, wb):
    grid = (NPAD // BN,)
    blk = pl.BlockSpec((BN, HID), lambda i: (i, 0))
    blkx = pl.BlockSpec((4, BN), lambda i: (0, i))
    blka = pl.BlockSpec((NC, 4, BN), lambda i: (0, 0, i))
    return pl.pallas_call(
        _node_mid_body,
        grid=grid,
        in_specs=[blk, blk, blk, blka, blkx,
                  _full((HID, HID)), _full((HID, HID)), _full((1, HID)),
                  _full((HID, HID)), _full((1, HID)),
                  _full((HID, HID)), _full((HID, HID))],
        out_specs=[blk, blkx, blk, blk],
        out_shape=[jax.ShapeDtypeStruct((NPAD, HID), F32),
                   jax.ShapeDtypeStruct((4, NPAD), F32),
                   jax.ShapeDtypeStruct((NPAD, HID), F32),
                   jax.ShapeDtypeStruct((NPAD, HID), F32)],
    )(h, m0, m1, a4, xt, w1h, w1m, nb1, w2, nb2, wa, wb)


def _node_fin(h, m0, m1, a4, xt, w1h, w1m, nb1, w2, nb2, wo, bo):
    grid = (NPAD // BN,)
    blk = pl.BlockSpec((BN, HID), lambda i: (i, 0))
    blkx = pl.BlockSpec((4, BN), lambda i: (0, i))
    blka = pl.BlockSpec((NC, 4, BN), lambda i: (0, 0, i))
    return pl.pallas_call(
        _node_fin_body,
        grid=grid,
        in_specs=[blk, blk, blk, blka, blkx,
                  _full((HID, HID)), _full((HID, HID)), _full((1, HID)),
                  _full((HID, HID)), _full((1, HID)),
                  _full((HID, HID)), _full((1, HID))],
        out_specs=[blk, blkx],
        out_shape=[jax.ShapeDtypeStruct((NPAD, HID), F32),
                   jax.ShapeDtypeStruct((4, NPAD), F32)],
    )(h, m0, m1, a4, xt, w1h, w1m, nb1, w2, nb2, wo, bo)


def _sc_gather(tha, thb, xf, rown, coln):
    mesh = plsc.VectorSubcoreMesh(core_axis_name="c", subcore_axis_name="s")
    fn = pl.kernel(
        _gather_body,
        out_type=[jax.ShapeDtypeStruct((ROWS, 128, HID), F32),
                  jax.ShapeDtypeStruct((ROWS, 128, HID), F32),
                  jax.ShapeDtypeStruct((ROWS, 128), F32),
                  jax.ShapeDtypeStruct((ROWS, 128), F32),
                  jax.ShapeDtypeStruct((ROWS, 128), F32),
                  jax.ShapeDtypeStruct((ROWS, 128), F32)],
        mesh=mesh,
        scratch_types=[pltpu.VMEM((2, 1, 128), jnp.int32),
                       pltpu.VMEM((2, 1, 128), jnp.int32),
                       pltpu.VMEM((2, 6, 128), jnp.int32),
                       pltpu.VMEM((2, 6, 128), F32),
                       pltpu.VMEM((2, 128, HID), F32),
                       pltpu.VMEM((2, 128, HID), F32),
                       pltpu.VMEM((2, 3, 128), F32),
                       pltpu.VMEM((2, 128), F32),
                       pltpu.SemaphoreType.DMA,
                       pltpu.SemaphoreType.DMA,
                       pltpu.SemaphoreType.DMA],
    )
    return fn(tha, thb, xf, rown, coln)


def _sc_scatter(zm, z4, rown, m3, t0, t1, t2):
    mesh = plsc.VectorSubcoreMesh(core_axis_name="c", subcore_axis_name="s")
    fn = pl.kernel(
        _scatter_body,
        out_type=[jax.ShapeDtypeStruct((NC, NPAD, HID), F32),
                  jax.ShapeDtypeStruct((NC, 4 * NPAD), F32)],
        mesh=mesh,
        scratch_types=[pltpu.VMEM((2, 1, 128), jnp.int32),
                       pltpu.VMEM((2, 128, HID), F32),
                       pltpu.VMEM((2, 3, 128), F32),
                       pltpu.VMEM((2, 4, 128), jnp.int32),
                       pltpu.VMEM((128,), F32),
                       pltpu.VMEM_SHARED((4 * NPAD,), F32),
                       pltpu.VMEM_SHARED((NPAD, HID), F32),
                       pltpu.SemaphoreType.DMA,
                       pltpu.SemaphoreType.DMA],
    )
    return fn(zm, z4, rown, m3, t0, t1, t2)


def kernel(complex_x, complex_pos, complex_edge_index, complex_edge_attr,
           emb_in_w, emb_in_b, emb_out_w, emb_out_b,
           edge_w1, edge_b1, edge_w2, edge_b2,
           node_w1, node_b1, node_w2, node_b2,
           coord_w1, coord_b1, coord_w2):
    row = complex_edge_index[0]
    col = complex_edge_index[1]
    pad_ids = (N + (jnp.arange(EPAD - E, dtype=jnp.int32) % (NPAD - N))
               ).astype(jnp.int32)
    rown = jnp.concatenate([row, pad_ids]).reshape(ROWS, 128)
    coln = jnp.concatenate([col, pad_ids]).reshape(ROWS, 128)
    x0 = jnp.pad(complex_pos.T, ((0, 1), (0, NPAD - N)))
    xin = jnp.pad(complex_x, ((0, NPAD - N), (0, 0)))
    eap = jnp.pad(complex_edge_attr, ((0, EPAD - E), (0, 0)))
    zm = jnp.zeros((NPAD // NS, HID), F32)
    z4 = jnp.zeros((4 * NPAD // NS,), F32)

    def wsplit(l):
        w1 = edge_w1[l]
        return (w1[:HID], w1[HID:2 * HID], w1[2 * HID:2 * HID + 1],
                w1[2 * HID + 1:])

    wa0, wb0, _, _ = wsplit(0)
    h, ha, hb = _k0(xin, emb_in_w, emb_in_b[None], wa0, wb0)
    x = x0
    for l in range(2):
        _, _, wr, we = wsplit(l)
        hag, hbg, gd0, gd1, gd2, radg = _sc_gather(
            ha, hb, x.reshape(4 * NPAD), rown, coln)
        m2, tr0, tr1, tr2 = _edge(
            hag.reshape(EPAD, HID), hbg.reshape(EPAD, HID),
            gd0.reshape(EPAD, 1), gd1.reshape(EPAD, 1), gd2.reshape(EPAD, 1),
            radg.reshape(EPAD, 1), eap,
            wr, we, edge_b1[l][None], edge_w2[l], edge_b2[l][None],
            coord_w1[l], coord_b1[l][None], coord_w2[l].reshape(1, HID))
        accm, acct = _sc_scatter(zm, z4, rown,
                                 m2.reshape(ROWS, 128, HID),
                                 tr0.reshape(ROWS, 128),
                                 tr1.reshape(ROWS, 128),
                                 tr2.reshape(ROWS, 128))
        acct = acct.reshape(NC, 4, NPAD)
        w1h = node_w1[l][:HID]
        w1m = node_w1[l][HID:]
        if l == 0:
            wa1, wb1, _, _ = wsplit(1)
            h, x, ha, hb = _node_mid(
                h, accm[0], accm[1], acct, x,
                w1h, w1m, node_b1[l][None], node_w2[l], node_b2[l][None],
                wa1, wb1)
        else:
            hout, x = _node_fin(
                h, accm[0], accm[1], acct, x,
                w1h, w1m, node_b1[l][None], node_w2[l], node_b2[l][None],
                emb_out_w, emb_out_b[None])
    return hout[:N], x[:3, :N].T


# speedup vs baseline: 2.8812x; 1.0017x over previous
import functools

import jax
import jax.numpy as jnp
from jax import lax
from jax.experimental import pallas as pl
from jax.experimental.pallas import tpu as pltpu
from jax.experimental.pallas import tpu_sc as plsc

N = 10000
NPAD = 10240
E = 320000
EPAD = 327680
ROWS = EPAD // 128
HID = 128
EDIM = 16
NC = 2
NS = 16
NW = NC * NS
RPW = ROWS // NW
ROWSH = ROWS // 2
EPADH = EPAD // 2
RPWH = ROWSH // NW
K = 2
NCHUNK = RPW // K
BE = 1024
BN = 1024
F32 = jnp.float32
BF16 = jnp.bfloat16


def _silu(x):
    return x * jax.nn.sigmoid(x)


def _k0_body(x_ref, wi_ref, bi_ref, wa_ref, wb_ref, h_ref, ha_ref, hb_ref):
    h = x_ref[...] @ wi_ref[...] + bi_ref[...]
    h_ref[...] = h
    ha_ref[...] = h @ wa_ref[...]
    hb_ref[...] = h @ wb_ref[...]


def _edge_body(ha_ref, hb_ref, d0_ref, d1_ref, d2_ref, rad_ref, ea_ref,
               wr_ref, we_ref, b1_ref, w2_ref, b2_ref,
               c1_ref, cb1_ref, c2r_ref, m_ref, t0_ref, t1o_ref, t2_ref):
    radial = rad_ref[...]
    t1 = (ha_ref[...] + hb_ref[...] + radial * wr_ref[...]
          + ea_ref[...] @ we_ref[...] + b1_ref[...])
    m1 = _silu(t1)
    m2 = _silu(m1 @ w2_ref[...] + b2_ref[...])
    cm = _silu(m2 @ c1_ref[...] + cb1_ref[...])
    m_ref[...] = m2
    sca = jnp.sum(cm * c2r_ref[...], axis=1, keepdims=True)
    t0_ref[...] = d0_ref[...] * sca
    t1o_ref[...] = d1_ref[...] * sca
    t2_ref[...] = d2_ref[...] * sca


def _coord_update(xt, a4):
    ct = jnp.sum(a4, axis=0)
    cnt = jnp.clip(ct[3:4, :], 1.0, None)
    comp = lax.broadcasted_iota(jnp.int32, ct.shape, 0)
    return xt + jnp.where(comp < 3, ct / cnt, 0.0)


def _node_mid_body(h_ref, m0_ref, m1_ref, a4_ref, xt_ref,
                   w1h_ref, w1m_ref, nb1_ref, w2_ref, nb2_ref,
                   wa_ref, wb_ref, hn_ref, xn_ref, ha_ref, hb_ref):
    magg = m0_ref[...] + m1_ref[...]
    o = _silu(h_ref[...] @ w1h_ref[...] + magg @ w1m_ref[...] + nb1_ref[...])
    o = o @ w2_ref[...] + nb2_ref[...]
    hn = h_ref[...] + o
    hn_ref[...] = hn
    xn_ref[...] = _coord_update(xt_ref[...], a4_ref[...])
    ha_ref[...] = hn @ wa_ref[...]
    hb_ref[...] = hn @ wb_ref[...]


def _node_fin_body(h_ref, m0_ref, m1_ref, a4_ref, xt_ref,
                   w1h_ref, w1m_ref, nb1_ref, w2_ref, nb2_ref,
                   wo_ref, bo_ref, hout_ref, xn_ref):
    magg = m0_ref[...] + m1_ref[...]
    o = _silu(h_ref[...] @ w1h_ref[...] + magg @ w1m_ref[...] + nb1_ref[...])
    o = o @ w2_ref[...] + nb2_ref[...]
    hn = h_ref[...] + o
    hout_ref[...] = hn @ wo_ref[...] + bo_ref[...]
    xn_ref[...] = _coord_update(xt_ref[...], a4_ref[...])


def _gather_body(tha, thb, xf, rown, coln, oha, ohb, od0, od1, od2, orad,
                 ridx, cidx, cix, gx, bha, bhb, bdd, brad,
                 semI, semG, semW):
    rpw = RPWH
    c = lax.axis_index("c")
    s = lax.axis_index("s")
    base = (s * NC + c) * rpw

    def fire_idx(slot, r):
        pltpu.async_copy(rown.at[pl.ds(r, 1)], ridx.at[slot], semI)
        pltpu.async_copy(coln.at[pl.ds(r, 1)], cidx.at[slot], semI)

    def drain_idx(slot, r):
        pltpu.make_async_copy(rown.at[pl.ds(r, 1)], ridx.at[slot],
                              semI).wait()
        pltpu.make_async_copy(coln.at[pl.ds(r, 1)], cidx.at[slot],
                              semI).wait()

    def compute_cix(slot):
        for t in range(8):
            sl = pl.ds(t * 16, 16)
            ir = ridx[slot, 0, sl]
            ic = cidx[slot, 0, sl]
            for comp in range(3):
                off = jnp.int32(comp * NPAD)
                cix[slot, comp, sl] = ir + off
                cix[slot, 3 + comp, sl] = ic + off

    def fire_gathers(slot):
        pltpu.async_copy(tha.at[ridx.at[slot, 0]], bha.at[slot], semG)
        pltpu.async_copy(thb.at[cidx.at[slot, 0]], bhb.at[slot], semG)
        for q in range(6):
            pltpu.async_copy(xf.at[cix.at[slot, q]], gx.at[slot, q], semG)

    def drain_gathers(slot):
        pltpu.make_async_copy(tha.at[ridx.at[slot, 0]], bha.at[slot],
                              semG).wait()
        pltpu.make_async_copy(thb.at[cidx.at[slot, 0]], bhb.at[slot],
                              semG).wait()
        for q in range(6):
            pltpu.make_async_copy(xf.at[cix.at[slot, q]], gx.at[slot, q],
                                  semG).wait()

    def compute_diff(slot):
        for t in range(8):
            sl = pl.ds(t * 16, 16)
            rad = jnp.zeros((16,), F32)
            for comp in range(3):
                dv = gx[slot, comp, sl] - gx[slot, 3 + comp, sl]
                bdd[slot, comp, sl] = dv
                rad = rad + dv * dv
            brad[slot, sl] = rad

    def fire_writes(slot, r):
        pltpu.async_copy(bha.at[slot], oha.at[r], semW)
        pltpu.async_copy(bhb.at[slot], ohb.at[r], semW)
        pltpu.async_copy(bdd.at[slot, 0], od0.at[r], semW)
        pltpu.async_copy(bdd.at[slot, 1], od1.at[r], semW)
        pltpu.async_copy(bdd.at[slot, 2], od2.at[r], semW)
        pltpu.async_copy(brad.at[slot], orad.at[r], semW)

    def drain_writes(slot, r):
        pltpu.make_async_copy(bha.at[slot], oha.at[r], semW).wait()
        pltpu.make_async_copy(bhb.at[slot], ohb.at[r], semW).wait()
        pltpu.make_async_copy(bdd.at[slot, 0], od0.at[r], semW).wait()
        pltpu.make_async_copy(bdd.at[slot, 1], od1.at[r], semW).wait()
        pltpu.make_async_copy(bdd.at[slot, 2], od2.at[r], semW).wait()
        pltpu.make_async_copy(brad.at[slot], orad.at[r], semW).wait()

    pltpu.sync_copy(rown.at[pl.ds(base, 1)], ridx.at[0])
    pltpu.sync_copy(coln.at[pl.ds(base, 1)], cidx.at[0])
    compute_cix(0)
    fire_gathers(0)
    fire_idx(1, base + 1)

    def pair(i, carry):
        for b in range(2):
            ci = 2 * i + b
            r0 = base + ci
            nb = 1 - b

            @pl.when(ci >= 1)
            def _():
                drain_writes(nb, r0 - 1)
            drain_idx(nb, base + ((ci + 1) % rpw))
            compute_cix(nb)
            fire_gathers(nb)
            drain_gathers(b)
            fire_idx(b, base + ((ci + 2) % rpw))
            compute_diff(b)
            fire_writes(b, r0)
        return carry

    lax.fori_loop(0, rpw // 2, pair, 0)
    drain_gathers(0)
    drain_idx(1, base)
    drain_writes(1, base + rpw - 1)


def _scatter_body(zm, z4, rowna, m3a, t0a, t1a, t2a, rownb, m3b, t0b, t1b,
                  t2b, outm, outt,
                  ridx, mbuf, tbuf, tix, ones, acc4, accm, semL, semS):
    c = lax.axis_index("c")
    s = lax.axis_index("s")
    base = (s * NC + c) * RPWH
    stripe = NPAD // NS
    stripe4 = 4 * NPAD // NS
    pltpu.sync_copy(zm, accm.at[pl.ds(s * stripe, stripe)])
    pltpu.sync_copy(z4, acc4.at[pl.ds(s * stripe4, stripe4)])
    for t in range(8):
        ones[pl.ds(t * 16, 16)] = jnp.ones((16,), F32)
    plsc.subcore_barrier()

    def run_half(rown, m3, t0, t1, t2):
        def fire_loads(slot, r):
            pltpu.async_copy(rown.at[pl.ds(r, 1)], ridx.at[slot], semL)
            pltpu.async_copy(m3.at[r], mbuf.at[slot], semL)
            pltpu.async_copy(t0.at[r], tbuf.at[slot, 0], semL)
            pltpu.async_copy(t1.at[r], tbuf.at[slot, 1], semL)
            pltpu.async_copy(t2.at[r], tbuf.at[slot, 2], semL)

        def drain_loads(slot, r):
            pltpu.make_async_copy(rown.at[pl.ds(r, 1)], ridx.at[slot],
                                  semL).wait()
            pltpu.make_async_copy(m3.at[r], mbuf.at[slot], semL).wait()
            pltpu.make_async_copy(t0.at[r], tbuf.at[slot, 0], semL).wait()
            pltpu.make_async_copy(t1.at[r], tbuf.at[slot, 1], semL).wait()
            pltpu.make_async_copy(t2.at[r], tbuf.at[slot, 2], semL).wait()

        def fire_scatters(slot):
            pltpu.async_copy(mbuf.at[slot], accm.at[ridx.at[slot, 0]], semS,
                             add=True)
            for q in range(3):
                pltpu.async_copy(tbuf.at[slot, q], acc4.at[tix.at[slot, q]],
                                 semS, add=True)
            pltpu.async_copy(ones, acc4.at[tix.at[slot, 3]], semS, add=True)

        def drain_scatters(slot):
            pltpu.make_async_copy(mbuf.at[slot], accm.at[ridx.at[slot, 0]],
                                  semS).wait()
            for q in range(3):
                pltpu.make_async_copy(tbuf.at[slot, q],
                                      acc4.at[tix.at[slot, q]], semS).wait()
            pltpu.make_async_copy(ones, acc4.at[tix.at[slot, 3]],
                                  semS).wait()

        fire_loads(0, base)

        def pair(i, carry):
            for b in range(2):
                ci = 2 * i + b
                nb = 1 - b

                @pl.when(ci >= 1)
                def _():
                    drain_scatters(nb)
                fire_loads(nb, base + ((ci + 1) % RPWH))
                drain_loads(b, base + ci)
                for t in range(8):
                    sl = pl.ds(t * 16, 16)
                    ir = ridx[b, 0, sl]
                    for q in range(4):
                        tix[b, q, sl] = ir + jnp.int32(q * NPAD)
                fire_scatters(b)
            return carry

        lax.fori_loop(0, RPWH // 2, pair, 0)
        drain_scatters(1)
        drain_loads(0, base)

    run_half(rowna, m3a, t0a, t1a, t2a)
    run_half(rownb, m3b, t0b, t1b, t2b)
    plsc.subcore_barrier()

    @pl.when(s == 0)
    def _():
        pltpu.sync_copy(accm, outm.at[c])
        pltpu.sync_copy(acc4, outt.at[c])


def _full(shape):
    return pl.BlockSpec(shape, lambda i: (0,) * len(shape))


def _k0(x, wi, bi, wa, wb):
    grid = (NPAD // BN,)
    blk = pl.BlockSpec((BN, HID), lambda i: (i, 0))
    return pl.pallas_call(
        _k0_body,
        grid=grid,
        in_specs=[blk, _full((HID, HID)), _full((1, HID)),
                  _full((HID, HID)), _full((HID, HID))],
        out_specs=[blk, blk, blk],
        out_shape=[jax.ShapeDtypeStruct((NPAD, HID), F32)] * 3,
    )(x, wi, bi, wa, wb)


def _edge(ha, hb, d0, d1, d2, rad, ea, wr, we, b1, w2, b2, c1, cb1, c2r):
    grid = (EPADH // BE,)
    blk = pl.BlockSpec((BE, HID), lambda i: (i, 0))
    blk1 = pl.BlockSpec((BE, 1), lambda i: (i, 0))
    blke = pl.BlockSpec((BE, EDIM), lambda i: (i, 0))
    return pl.pallas_call(
        _edge_body,
        grid=grid,
        in_specs=[blk, blk, blk1, blk1, blk1, blk1, blke,
                  _full((1, HID)), _full((EDIM, HID)), _full((1, HID)),
                  _full((HID, HID)), _full((1, HID)),
                  _full((HID, HID)), _full((1, HID)), _full((1, HID))],
        out_specs=[blk, blk1, blk1, blk1],
        out_shape=[jax.ShapeDtypeStruct((EPADH, HID), F32),
                   jax.ShapeDtypeStruct((EPADH, 1), F32),
                   jax.ShapeDtypeStruct((EPADH, 1), F32),
                   jax.ShapeDtypeStruct((EPADH, 1), F32)],
    )(ha, hb, d0, d1, d2, rad, ea, wr, we, b1, w2, b2, c1, cb1, c2r)


def _node_mid(h, m0, m1, a4, xt, w1h, w1m, nb1, w2, nb2, wa, wb):
    grid = (NPAD // BN,)
    blk = pl.BlockSpec((BN, HID), lambda i: (i, 0))
    blkx = pl.BlockSpec((4, BN), lambda i: (0, i))
    blka = pl.BlockSpec((NC, 4, BN), lambda i: (0, 0, i))
    return pl.pallas_call(
        _node_mid_body,
        grid=grid,
        in_specs=[blk, blk, blk, blka, blkx,
                  _full((HID, HID)), _full((HID, HID)), _full((1, HID)),
                  _full((HID, HID)), _full((1, HID)),
                  _full((HID, HID)), _full((HID, HID))],
        out_specs=[blk, blkx, blk, blk],
        out_shape=[jax.ShapeDtypeStruct((NPAD, HID), F32),
                   jax.ShapeDtypeStruct((4, NPAD), F32),
                   jax.ShapeDtypeStruct((NPAD, HID), F32),
                   jax.ShapeDtypeStruct((NPAD, HID), F32)],
    )(h, m0, m1, a4, xt, w1h, w1m, nb1, w2, nb2, wa, wb)


def _node_fin(h, m0, m1, a4, xt, w1h, w1m, nb1, w2, nb2, wo, bo):
    grid = (NPAD // BN,)
    blk = pl.BlockSpec((BN, HID), lambda i: (i, 0))
    blkx = pl.BlockSpec((4, BN), lambda i: (0, i))
    blka = pl.BlockSpec((NC, 4, BN), lambda i: (0, 0, i))
    return pl.pallas_call(
        _node_fin_body,
        grid=grid,
        in_specs=[blk, blk, blk, blka, blkx,
                  _full((HID, HID)), _full((HID, HID)), _full((1, HID)),
                  _full((HID, HID)), _full((1, HID)),
                  _full((HID, HID)), _full((1, HID))],
        out_specs=[blk, blkx],
        out_shape=[jax.ShapeDtypeStruct((NPAD, HID), F32),
                   jax.ShapeDtypeStruct((4, NPAD), F32)],
    )(h, m0, m1, a4, xt, w1h, w1m, nb1, w2, nb2, wo, bo)


def _sc_gather(tha, thb, xf, rown, coln):
    mesh = plsc.VectorSubcoreMesh(core_axis_name="c", subcore_axis_name="s")
    fn = pl.kernel(
        _gather_body,
        out_type=[jax.ShapeDtypeStruct((ROWSH, 128, HID), F32),
                  jax.ShapeDtypeStruct((ROWSH, 128, HID), F32),
                  jax.ShapeDtypeStruct((ROWSH, 128), F32),
                  jax.ShapeDtypeStruct((ROWSH, 128), F32),
                  jax.ShapeDtypeStruct((ROWSH, 128), F32),
                  jax.ShapeDtypeStruct((ROWSH, 128), F32)],
        mesh=mesh,
        scratch_types=[pltpu.VMEM((2, 1, 128), jnp.int32),
                       pltpu.VMEM((2, 1, 128), jnp.int32),
                       pltpu.VMEM((2, 6, 128), jnp.int32),
                       pltpu.VMEM((2, 6, 128), F32),
                       pltpu.VMEM((2, 128, HID), F32),
                       pltpu.VMEM((2, 128, HID), F32),
                       pltpu.VMEM((2, 3, 128), F32),
                       pltpu.VMEM((2, 128), F32),
                       pltpu.SemaphoreType.DMA,
                       pltpu.SemaphoreType.DMA,
                       pltpu.SemaphoreType.DMA],
    )
    return fn(tha, thb, xf, rown, coln)


def _sc_scatter(zm, z4, rowna, m3a, t0a, t1a, t2a, rownb, m3b, t0b, t1b,
                t2b):
    mesh = plsc.VectorSubcoreMesh(core_axis_name="c", subcore_axis_name="s")
    fn = pl.kernel(
        _scatter_body,
        out_type=[jax.ShapeDtypeStruct((NC, NPAD, HID), F32),
                  jax.ShapeDtypeStruct((NC, 4 * NPAD), F32)],
        mesh=mesh,
        scratch_types=[pltpu.VMEM((2, 1, 128), jnp.int32),
                       pltpu.VMEM((2, 128, HID), F32),
                       pltpu.VMEM((2, 3, 128), F32),
                       pltpu.VMEM((2, 4, 128), jnp.int32),
                       pltpu.VMEM((128,), F32),
                       pltpu.VMEM_SHARED((4 * NPAD,), F32),
                       pltpu.VMEM_SHARED((NPAD, HID), F32),
                       pltpu.SemaphoreType.DMA,
                       pltpu.SemaphoreType.DMA],
    )
    return fn(zm, z4, rowna, m3a, t0a, t1a, t2a, rownb, m3b, t0b, t1b, t2b)


def kernel(complex_x, complex_pos, complex_edge_index, complex_edge_attr,
           emb_in_w, emb_in_b, emb_out_w, emb_out_b,
           edge_w1, edge_b1, edge_w2, edge_b2,
           node_w1, node_b1, node_w2, node_b2,
           coord_w1, coord_b1, coord_w2):
    row = complex_edge_index[0]
    col = complex_edge_index[1]
    pad_ids = (N + (jnp.arange(EPAD - E, dtype=jnp.int32) % (NPAD - N))
               ).astype(jnp.int32)
    rown = jnp.concatenate([row, pad_ids]).reshape(ROWS, 128)
    coln = jnp.concatenate([col, pad_ids]).reshape(ROWS, 128)
    x0 = jnp.pad(complex_pos.T, ((0, 1), (0, NPAD - N)))
    xin = jnp.pad(complex_x, ((0, NPAD - N), (0, 0)))
    eap = jnp.pad(complex_edge_attr, ((0, EPAD - E), (0, 0)))
    zm = jnp.zeros((NPAD // NS, HID), F32)
    z4 = jnp.zeros((4 * NPAD // NS,), F32)

    def wsplit(l):
        w1 = edge_w1[l]
        return (w1[:HID], w1[HID:2 * HID], w1[2 * HID:2 * HID + 1],
                w1[2 * HID + 1:])

    wa0, wb0, _, _ = wsplit(0)
    h, ha, hb = _k0(xin, emb_in_w, emb_in_b[None], wa0, wb0)
    x = x0
    rh = (rown[:ROWSH], rown[ROWSH:])
    ch = (coln[:ROWSH], coln[ROWSH:])
    eah = (eap[:EPADH], eap[EPADH:])
    for l in range(2):
        _, _, wr, we = wsplit(l)
        xfl = x.reshape(4 * NPAD)
        ew = (wr, we, edge_b1[l][None], edge_w2[l], edge_b2[l][None],
              coord_w1[l], coord_b1[l][None], coord_w2[l].reshape(1, HID))
        halves = []
        gathered = [_sc_gather(ha, hb, xfl, rh[p], ch[p]) for p in range(2)]
        for p in range(2):
            hag, hbg, gd0, gd1, gd2, radg = gathered[p]
            m2, tr0, tr1, tr2 = _edge(
                hag.reshape(EPADH, HID), hbg.reshape(EPADH, HID),
                gd0.reshape(EPADH, 1), gd1.reshape(EPADH, 1),
                gd2.reshape(EPADH, 1), radg.reshape(EPADH, 1), eah[p], *ew)
            halves.append((rh[p], m2.reshape(ROWSH, 128, HID),
                           tr0.reshape(ROWSH, 128), tr1.reshape(ROWSH, 128),
                           tr2.reshape(ROWSH, 128)))
        accm, acct = _sc_scatter(zm, z4, *halves[0], *halves[1])
        acct = acct.reshape(NC, 4, NPAD)
        w1h = node_w1[l][:HID]
        w1m = node_w1[l][HID:]
        if l == 0:
            wa1, wb1, _, _ = wsplit(1)
            h, x, ha, hb = _node_mid(
                h, accm[0], accm[1], acct, x,
                w1h, w1m, node_b1[l][None], node_w2[l], node_b2[l][None],
                wa1, wb1)
        else:
            hout, x = _node_fin(
                h, accm[0], accm[1], acct, x,
                w1h, w1m, node_b1[l][None], node_w2[l], node_b2[l][None],
                emb_out_w, emb_out_b[None])
    return hout[:N], x[:3, :N].T


# speedup vs baseline: 2.8829x; 1.0006x over previous
import jax
import jax.numpy as jnp
from jax import lax
from jax.experimental import pallas as pl
from jax.experimental.pallas import tpu as pltpu
from jax.experimental.pallas import tpu_sc as plsc

N = 10000
NPAD = 10240
E = 320000
EPAD = 327680
ROWS = EPAD // 128
HID = 128
EDIM = 16
NC = 2
NS = 16
NW = NC * NS
ROWSH = ROWS // 2
EPADH = EPAD // 2
RPWH = ROWSH // NW
BE = 1024
BN = 1024
F32 = jnp.float32


def _silu(x):
    return x * jax.nn.sigmoid(x)


def _k0_body(x_ref, wi_ref, bi_ref, wa_ref, wb_ref, h_ref, ha_ref, hb_ref):
    h = x_ref[...] @ wi_ref[...] + bi_ref[...]
    h_ref[...] = h
    ha_ref[...] = h @ wa_ref[...]
    hb_ref[...] = h @ wb_ref[...]


def _edge_body(ha_ref, hb_ref, d0_ref, d1_ref, d2_ref, rad_ref, ea_ref,
               wr_ref, we_ref, b1_ref, w2_ref, b2_ref,
               c1_ref, cb1_ref, c2r_ref, m_ref, t0_ref, t1o_ref, t2_ref):
    radial = rad_ref[...]
    t1 = (ha_ref[...] + hb_ref[...] + radial * wr_ref[...]
          + ea_ref[...] @ we_ref[...] + b1_ref[...])
    m1 = _silu(t1)
    m2 = _silu(m1 @ w2_ref[...] + b2_ref[...])
    cm = _silu(m2 @ c1_ref[...] + cb1_ref[...])
    m_ref[...] = m2
    sca = jnp.sum(cm * c2r_ref[...], axis=1, keepdims=True)
    t0_ref[...] = d0_ref[...] * sca
    t1o_ref[...] = d1_ref[...] * sca
    t2_ref[...] = d2_ref[...] * sca


def _coord_update(xt, a4):
    ct = jnp.sum(a4, axis=0)
    cnt = jnp.clip(ct[3:4, :], 1.0, None)
    comp = lax.broadcasted_iota(jnp.int32, ct.shape, 0)
    return xt + jnp.where(comp < 3, ct / cnt, 0.0)


def _node_mid_body(h_ref, m0_ref, m1_ref, a4_ref, xt_ref,
                   w1h_ref, w1m_ref, nb1_ref, w2_ref, nb2_ref,
                   wa_ref, wb_ref, hn_ref, xn_ref, ha_ref, hb_ref):
    magg = m0_ref[...] + m1_ref[...]
    o = _silu(h_ref[...] @ w1h_ref[...] + magg @ w1m_ref[...] + nb1_ref[...])
    o = o @ w2_ref[...] + nb2_ref[...]
    hn = h_ref[...] + o
    hn_ref[...] = hn
    xn_ref[...] = _coord_update(xt_ref[...], a4_ref[...])
    ha_ref[...] = hn @ wa_ref[...]
    hb_ref[...] = hn @ wb_ref[...]


def _node_fin_body(h_ref, m0_ref, m1_ref, a4_ref, xt_ref,
                   w1h_ref, w1m_ref, nb1_ref, w2_ref, nb2_ref,
                   wo_ref, bo_ref, hout_ref, xn_ref):
    magg = m0_ref[...] + m1_ref[...]
    o = _silu(h_ref[...] @ w1h_ref[...] + magg @ w1m_ref[...] + nb1_ref[...])
    o = o @ w2_ref[...] + nb2_ref[...]
    hn = h_ref[...] + o
    hout_ref[...] = hn @ wo_ref[...] + bo_ref[...]
    xn_ref[...] = _coord_update(xt_ref[...], a4_ref[...])


def _gather_body(tha, thb, xf, rown, coln, oha, ohb, od0, od1, od2, orad,
                 ridx, cidx, cix, gx, bha, bhb, bdd, brad,
                 semI, semG, semW):
    rpw = RPWH
    c = lax.axis_index("c")
    s = lax.axis_index("s")
    base = (s * NC + c) * rpw

    def fire_idx(slot, r):
        pltpu.async_copy(rown.at[pl.ds(r, 1)], ridx.at[slot], semI)
        pltpu.async_copy(coln.at[pl.ds(r, 1)], cidx.at[slot], semI)

    def drain_idx(slot, r):
        pltpu.make_async_copy(rown.at[pl.ds(r, 1)], ridx.at[slot],
                              semI).wait()
        pltpu.make_async_copy(coln.at[pl.ds(r, 1)], cidx.at[slot],
                              semI).wait()

    def compute_cix(slot):
        for t in range(8):
            sl = pl.ds(t * 16, 16)
            ir = ridx[slot, 0, sl]
            ic = cidx[slot, 0, sl]
            for comp in range(3):
                off = jnp.int32(comp * NPAD)
                cix[slot, comp, sl] = ir + off
                cix[slot, 3 + comp, sl] = ic + off

    def fire_gathers(slot):
        pltpu.async_copy(tha.at[ridx.at[slot, 0]], bha.at[slot], semG)
        pltpu.async_copy(thb.at[cidx.at[slot, 0]], bhb.at[slot], semG)
        for q in range(6):
            pltpu.async_copy(xf.at[cix.at[slot, q]], gx.at[slot, q], semG)

    def drain_gathers(slot):
        pltpu.make_async_copy(tha.at[ridx.at[slot, 0]], bha.at[slot],
                              semG).wait()
        pltpu.make_async_copy(thb.at[cidx.at[slot, 0]], bhb.at[slot],
                              semG).wait()
        for q in range(6):
            pltpu.make_async_copy(xf.at[cix.at[slot, q]], gx.at[slot, q],
                                  semG).wait()

    def compute_diff(slot):
        for t in range(8):
            sl = pl.ds(t * 16, 16)
            rad = jnp.zeros((16,), F32)
            for comp in range(3):
                dv = gx[slot, comp, sl] - gx[slot, 3 + comp, sl]
                bdd[slot, comp, sl] = dv
                rad = rad + dv * dv
            brad[slot, sl] = rad

    def fire_writes(slot, r):
        pltpu.async_copy(bha.at[slot], oha.at[r], semW)
        pltpu.async_copy(bhb.at[slot], ohb.at[r], semW)
        pltpu.async_copy(bdd.at[slot, 0], od0.at[r], semW)
        pltpu.async_copy(bdd.at[slot, 1], od1.at[r], semW)
        pltpu.async_copy(bdd.at[slot, 2], od2.at[r], semW)
        pltpu.async_copy(brad.at[slot], orad.at[r], semW)

    def drain_writes(slot, r):
        pltpu.make_async_copy(bha.at[slot], oha.at[r], semW).wait()
        pltpu.make_async_copy(bhb.at[slot], ohb.at[r], semW).wait()
        pltpu.make_async_copy(bdd.at[slot, 0], od0.at[r], semW).wait()
        pltpu.make_async_copy(bdd.at[slot, 1], od1.at[r], semW).wait()
        pltpu.make_async_copy(bdd.at[slot, 2], od2.at[r], semW).wait()
        pltpu.make_async_copy(brad.at[slot], orad.at[r], semW).wait()

    pltpu.sync_copy(rown.at[pl.ds(base, 1)], ridx.at[0])
    pltpu.sync_copy(coln.at[pl.ds(base, 1)], cidx.at[0])
    compute_cix(0)
    fire_gathers(0)
    fire_idx(1, base + 1)

    def pair(i, carry):
        for b in range(2):
            ci = 2 * i + b
            r0 = base + ci
            nb = 1 - b

            @pl.when(ci >= 1)
            def _():
                drain_writes(nb, r0 - 1)
            drain_idx(nb, base + ((ci + 1) % rpw))
            compute_cix(nb)
            fire_gathers(nb)
            drain_gathers(b)
            fire_idx(b, base + ((ci + 2) % rpw))
            compute_diff(b)
            fire_writes(b, r0)
        return carry

    lax.fori_loop(0, rpw // 2, pair, 0)
    drain_gathers(0)
    drain_idx(1, base)
    drain_writes(1, base + rpw - 1)


def _scatter_body(zm, z4, rowna, m3a, t0a, t1a, t2a, rownb, m3b, t0b, t1b,
                  t2b, outm, outt,
                  ridx, mbuf, tbuf, tix, ones, acc4, accm, semL, semS):
    c = lax.axis_index("c")
    s = lax.axis_index("s")
    base = (s * NC + c) * RPWH
    stripe = NPAD // NS
    stripe4 = 4 * NPAD // NS
    pltpu.sync_copy(zm, accm.at[pl.ds(s * stripe, stripe)])
    pltpu.sync_copy(z4, acc4.at[pl.ds(s * stripe4, stripe4)])
    for t in range(8):
        ones[pl.ds(t * 16, 16)] = jnp.ones((16,), F32)
    plsc.subcore_barrier()

    def run_half(rown, m3, t0, t1, t2):
        def fire_loads(slot, r):
            pltpu.async_copy(rown.at[pl.ds(r, 1)], ridx.at[slot], semL)
            pltpu.async_copy(m3.at[r], mbuf.at[slot], semL)
            pltpu.async_copy(t0.at[r], tbuf.at[slot, 0], semL)
            pltpu.async_copy(t1.at[r], tbuf.at[slot, 1], semL)
            pltpu.async_copy(t2.at[r], tbuf.at[slot, 2], semL)

        def drain_loads(slot, r):
            pltpu.make_async_copy(rown.at[pl.ds(r, 1)], ridx.at[slot],
                                  semL).wait()
            pltpu.make_async_copy(m3.at[r], mbuf.at[slot], semL).wait()
            pltpu.make_async_copy(t0.at[r], tbuf.at[slot, 0], semL).wait()
            pltpu.make_async_copy(t1.at[r], tbuf.at[slot, 1], semL).wait()
            pltpu.make_async_copy(t2.at[r], tbuf.at[slot, 2], semL).wait()

        def fire_scatters(slot):
            pltpu.async_copy(mbuf.at[slot], accm.at[ridx.at[slot, 0]], semS,
                             add=True)
            for q in range(3):
                pltpu.async_copy(tbuf.at[slot, q], acc4.at[tix.at[slot, q]],
                                 semS, add=True)
            pltpu.async_copy(ones, acc4.at[tix.at[slot, 3]], semS, add=True)

        def drain_scatters(slot):
            pltpu.make_async_copy(mbuf.at[slot], accm.at[ridx.at[slot, 0]],
                                  semS).wait()
            for q in range(3):
                pltpu.make_async_copy(tbuf.at[slot, q],
                                      acc4.at[tix.at[slot, q]], semS).wait()
            pltpu.make_async_copy(ones, acc4.at[tix.at[slot, 3]],
                                  semS).wait()

        fire_loads(0, base)

        def pair(i, carry):
            for b in range(2):
                ci = 2 * i + b
                nb = 1 - b

                @pl.when(ci >= 1)
                def _():
                    drain_scatters(nb)
                fire_loads(nb, base + ((ci + 1) % RPWH))
                drain_loads(b, base + ci)
                for t in range(8):
                    sl = pl.ds(t * 16, 16)
                    ir = ridx[b, 0, sl]
                    for q in range(4):
                        tix[b, q, sl] = ir + jnp.int32(q * NPAD)
                fire_scatters(b)
            return carry

        lax.fori_loop(0, RPWH // 2, pair, 0)
        drain_scatters(1)
        drain_loads(0, base)

    run_half(rowna, m3a, t0a, t1a, t2a)
    run_half(rownb, m3b, t0b, t1b, t2b)
    plsc.subcore_barrier()

    @pl.when(s == 0)
    def _():
        pltpu.sync_copy(accm, outm.at[c])
        pltpu.sync_copy(acc4, outt.at[c])


def _full(shape):
    return pl.BlockSpec(shape, lambda i: (0,) * len(shape))


def _k0(x, wi, bi, wa, wb):
    grid = (NPAD // BN,)
    blk = pl.BlockSpec((BN, HID), lambda i: (i, 0))
    return pl.pallas_call(
        _k0_body,
        grid=grid,
        in_specs=[blk, _full((HID, HID)), _full((1, HID)),
                  _full((HID, HID)), _full((HID, HID))],
        out_specs=[blk, blk, blk],
        out_shape=[jax.ShapeDtypeStruct((NPAD, HID), F32)] * 3,
    )(x, wi, bi, wa, wb)


def _edge(ha, hb, d0, d1, d2, rad, ea, wr, we, b1, w2, b2, c1, cb1, c2r):
    grid = (EPADH // BE,)
    blk = pl.BlockSpec((BE, HID), lambda i: (i, 0))
    blk1 = pl.BlockSpec((BE, 1), lambda i: (i, 0))
    blke = pl.BlockSpec((BE, EDIM), lambda i: (i, 0))
    return pl.pallas_call(
        _edge_body,
        grid=grid,
        in_specs=[blk, blk, blk1, blk1, blk1, blk1, blke,
                  _full((1, HID)), _full((EDIM, HID)), _full((1, HID)),
                  _full((HID, HID)), _full((1, HID)),
                  _full((HID, HID)), _full((1, HID)), _full((1, HID))],
        out_specs=[blk, blk1, blk1, blk1],
        out_shape=[jax.ShapeDtypeStruct((EPADH, HID), F32),
                   jax.ShapeDtypeStruct((EPADH, 1), F32),
                   jax.ShapeDtypeStruct((EPADH, 1), F32),
                   jax.ShapeDtypeStruct((EPADH, 1), F32)],
    )(ha, hb, d0, d1, d2, rad, ea, wr, we, b1, w2, b2, c1, cb1, c2r)


def _node_mid(h, m0, m1, a4, xt, w1h, w1m, nb1, w2, nb2, wa, wb):
    grid = (NPAD // BN,)
    blk = pl.BlockSpec((BN, HID), lambda i: (i, 0))
    blkx = pl.BlockSpec((4, BN), lambda i: (0, i))
    blka = pl.BlockSpec((NC, 4, BN), lambda i: (0, 0, i))
    return pl.pallas_call(
        _node_mid_body,
        grid=grid,
        in_specs=[blk, blk, blk, blka, blkx,
                  _full((HID, HID)), _full((HID, HID)), _full((1, HID)),
                  _full((HID, HID)), _full((1, HID)),
                  _full((HID, HID)), _full((HID, HID))],
        out_specs=[blk, blkx, blk, blk],
        out_shape=[jax.ShapeDtypeStruct((NPAD, HID), F32),
                   jax.ShapeDtypeStruct((4, NPAD), F32),
                   jax.ShapeDtypeStruct((NPAD, HID), F32),
                   jax.ShapeDtypeStruct((NPAD, HID), F32)],
    )(h, m0, m1, a4, xt, w1h, w1m, nb1, w2, nb2, wa, wb)


def _node_fin(h, m0, m1, a4, xt, w1h, w1m, nb1, w2, nb2, wo, bo):
    grid = (NPAD // BN,)
    blk = pl.BlockSpec((BN, HID), lambda i: (i, 0))
    blkx = pl.BlockSpec((4, BN), lambda i: (0, i))
    blka = pl.BlockSpec((NC, 4, BN), lambda i: (0, 0, i))
    return pl.pallas_call(
        _node_fin_body,
        grid=grid,
        in_specs=[blk, blk, blk, blka, blkx,
                  _full((HID, HID)), _full((HID, HID)), _full((1, HID)),
                  _full((HID, HID)), _full((1, HID)),
                  _full((HID, HID)), _full((1, HID))],
        out_specs=[blk, blkx],
        out_shape=[jax.ShapeDtypeStruct((NPAD, HID), F32),
                   jax.ShapeDtypeStruct((4, NPAD), F32)],
    )(h, m0, m1, a4, xt, w1h, w1m, nb1, w2, nb2, wo, bo)


def _sc_gather(tha, thb, xf, rown, coln):
    mesh = plsc.VectorSubcoreMesh(core_axis_name="c", subcore_axis_name="s")
    fn = pl.kernel(
        _gather_body,
        out_type=[jax.ShapeDtypeStruct((ROWSH, 128, HID), F32),
                  jax.ShapeDtypeStruct((ROWSH, 128, HID), F32),
                  jax.ShapeDtypeStruct((ROWSH, 128), F32),
                  jax.ShapeDtypeStruct((ROWSH, 128), F32),
                  jax.ShapeDtypeStruct((ROWSH, 128), F32),
                  jax.ShapeDtypeStruct((ROWSH, 128), F32)],
        mesh=mesh,
        scratch_types=[pltpu.VMEM((2, 1, 128), jnp.int32),
                       pltpu.VMEM((2, 1, 128), jnp.int32),
                       pltpu.VMEM((2, 6, 128), jnp.int32),
                       pltpu.VMEM((2, 6, 128), F32),
                       pltpu.VMEM((2, 128, HID), F32),
                       pltpu.VMEM((2, 128, HID), F32),
                       pltpu.VMEM((2, 3, 128), F32),
                       pltpu.VMEM((2, 128), F32),
                       pltpu.SemaphoreType.DMA,
                       pltpu.SemaphoreType.DMA,
                       pltpu.SemaphoreType.DMA],
    )
    return fn(tha, thb, xf, rown, coln)


def _sc_scatter(zm, z4, rowna, m3a, t0a, t1a, t2a, rownb, m3b, t0b, t1b,
                t2b):
    mesh = plsc.VectorSubcoreMesh(core_axis_name="c", subcore_axis_name="s")
    fn = pl.kernel(
        _scatter_body,
        out_type=[jax.ShapeDtypeStruct((NC, NPAD, HID), F32),
                  jax.ShapeDtypeStruct((NC, 4 * NPAD), F32)],
        mesh=mesh,
        scratch_types=[pltpu.VMEM((2, 1, 128), jnp.int32),
                       pltpu.VMEM((2, 128, HID), F32),
                       pltpu.VMEM((2, 3, 128), F32),
                       pltpu.VMEM((2, 4, 128), jnp.int32),
                       pltpu.VMEM((128,), F32),
                       pltpu.VMEM_SHARED((4 * NPAD,), F32),
                       pltpu.VMEM_SHARED((NPAD, HID), F32),
                       pltpu.SemaphoreType.DMA,
                       pltpu.SemaphoreType.DMA],
    )
    return fn(zm, z4, rowna, m3a, t0a, t1a, t2a, rownb, m3b, t0b, t1b, t2b)


def kernel(complex_x, complex_pos, complex_edge_index, complex_edge_attr,
           emb_in_w, emb_in_b, emb_out_w, emb_out_b,
           edge_w1, edge_b1, edge_w2, edge_b2,
           node_w1, node_b1, node_w2, node_b2,
           coord_w1, coord_b1, coord_w2):
    row = complex_edge_index[0]
    col = complex_edge_index[1]
    pad_ids = (N + (jnp.arange(EPAD - E, dtype=jnp.int32) % (NPAD - N))
               ).astype(jnp.int32)
    rown = jnp.concatenate([row, pad_ids]).reshape(ROWS, 128)
    coln = jnp.concatenate([col, pad_ids]).reshape(ROWS, 128)
    x0 = jnp.pad(complex_pos.T, ((0, 1), (0, NPAD - N)))
    xin = jnp.pad(complex_x, ((0, NPAD - N), (0, 0)))
    eap = jnp.pad(complex_edge_attr, ((0, EPAD - E), (0, 0)))
    zm = jnp.zeros((NPAD // NS, HID), F32)
    z4 = jnp.zeros((4 * NPAD // NS,), F32)

    def wsplit(l):
        w1 = edge_w1[l]
        return (w1[:HID], w1[HID:2 * HID], w1[2 * HID:2 * HID + 1],
                w1[2 * HID + 1:])

    wa0, wb0, _, _ = wsplit(0)
    h, ha, hb = _k0(xin, emb_in_w, emb_in_b[None], wa0, wb0)
    x = x0
    rh = (rown[:ROWSH], rown[ROWSH:])
    ch = (coln[:ROWSH], coln[ROWSH:])
    eah = (eap[:EPADH], eap[EPADH:])
    for l in range(2):
        _, _, wr, we = wsplit(l)
        xfl = x.reshape(4 * NPAD)
        ew = (wr, we, edge_b1[l][None], edge_w2[l], edge_b2[l][None],
              coord_w1[l], coord_b1[l][None], coord_w2[l].reshape(1, HID))
        halves = []
        gathered = [_sc_gather(ha, hb, xfl, rh[p], ch[p]) for p in range(2)]
        for p in range(2):
            hag, hbg, gd0, gd1, gd2, radg = gathered[p]
            m2, tr0, tr1, tr2 = _edge(
                hag.reshape(EPADH, HID), hbg.reshape(EPADH, HID),
                gd0.reshape(EPADH, 1), gd1.reshape(EPADH, 1),
                gd2.reshape(EPADH, 1), radg.reshape(EPADH, 1), eah[p], *ew)
            halves.append((rh[p], m2.reshape(ROWSH, 128, HID),
                           tr0.reshape(ROWSH, 128), tr1.reshape(ROWSH, 128),
                           tr2.reshape(ROWSH, 128)))
        accm, acct = _sc_scatter(zm, z4, *halves[0], *halves[1])
        acct = acct.reshape(NC, 4, NPAD)
        w1h = node_w1[l][:HID]
        w1m = node_w1[l][HID:]
        if l == 0:
            wa1, wb1, _, _ = wsplit(1)
            h, x, ha, hb = _node_mid(
                h, accm[0], accm[1], acct, x,
                w1h, w1m, node_b1[l][None], node_w2[l], node_b2[l][None],
                wa1, wb1)
        else:
            hout, x = _node_fin(
                h, accm[0], accm[1], acct, x,
                w1h, w1m, node_b1[l][None], node_w2[l], node_b2[l][None],
                emb_out_w, emb_out_b[None])
    return hout[:N], x[:3, :N].T
